# baseline (speedup 1.0000x reference)
import functools

import jax
import jax.numpy as jnp
from jax import lax
from jax.experimental import pallas as pl
from jax.experimental.pallas import tpu as pltpu
from jax.experimental.pallas import tpu_sc as plsc

_BASE_LEN = 2048
_CH = 24
_NBUF = 5
_DEPTH = 3
_LAG = 2
_BLK = 1032


def _copy_range(src, dst, b, r0, n_rows, bufs, in_sems, out_sems):
    n_chunks = n_rows // _CH
    in_h = [None] * n_chunks
    out_h = [None] * n_chunks
    for t in range(min(_DEPTH, n_chunks)):
        in_h[t] = pltpu.async_copy(
            src.at[b, pl.ds(r0 + t * _CH, _CH)], bufs[t % _NBUF], in_sems[t % _NBUF]
        )
    for t in range(n_chunks):
        if t - _LAG >= 0:
            out_h[t - _LAG].wait()
        nxt = t + _DEPTH
        if nxt < n_chunks:
            in_h[nxt] = pltpu.async_copy(
                src.at[b, pl.ds(r0 + nxt * _CH, _CH)],
                bufs[nxt % _NBUF],
                in_sems[nxt % _NBUF],
            )
        in_h[t].wait()
        out_h[t] = pltpu.async_copy(
            bufs[t % _NBUF], dst.at[b, pl.ds(r0 + t * _CH, _CH)], out_sems[t % _NBUF]
        )
    for t in range(max(0, n_chunks - _LAG), n_chunks):
        out_h[t].wait()


def _sc_bulk_body(vc, vo, *scratch):
    bufs = scratch[:_NBUF]
    in_sems = scratch[_NBUF:2 * _NBUF]
    out_sems = scratch[2 * _NBUF:3 * _NBUF]
    out_len = vo.shape[1]
    half_rows = out_len // 2

    c = lax.axis_index("c")
    s = lax.axis_index("s")
    b = s
    r0 = c * half_rows

    _copy_range(vc, vo, b, r0, half_rows, bufs, in_sems, out_sems)


def _tc_bulk_body(kc, ko):
    ko[...] = kc[...]


def _dma_fan_body(kc, vc, ko, vo, sk, sv):
    bs = ko.shape[0]
    out_len = ko.shape[1]
    hs = []
    for b in range(bs):
        h = pltpu.make_async_copy(kc.at[b, :out_len], ko.at[b], sk)
        h.start()
        hs.append(h)
        h = pltpu.make_async_copy(vc.at[b, :out_len], vo.at[b], sv)
        h.start()
        hs.append(h)
    for h in hs:
        h.wait()


def _patch_body(pos_ref, kv, vv, _ka, _va, ko, vo, sk, sv):
    seq = kv.shape[1]
    pos = pos_ref[0]
    ck = pltpu.make_async_copy(kv, ko.at[:, pl.ds(pos, seq)], sk)
    cv = pltpu.make_async_copy(vv, vo.at[:, pl.ds(pos, seq)], sv)
    ck.start()
    cv.start()
    ck.wait()
    cv.wait()


def kernel(k_cache, v_cache, input_pos, k_val, v_val):
    bs, seq, n_heads, head_dim = k_val.shape
    out_len = _BASE_LEN + seq
    pos = jnp.asarray(input_pos, dtype=jnp.int32).reshape(1)
    out_sd = jax.ShapeDtypeStruct((bs, out_len, n_heads, head_dim), k_cache.dtype)

    k_bulk, v_bulk = pl.pallas_call(
        _dma_fan_body,
        out_shape=(out_sd, out_sd),
        in_specs=[
            pl.BlockSpec(memory_space=pl.ANY),
            pl.BlockSpec(memory_space=pl.ANY),
        ],
        out_specs=(
            pl.BlockSpec(memory_space=pl.ANY),
            pl.BlockSpec(memory_space=pl.ANY),
        ),
        scratch_shapes=[pltpu.SemaphoreType.DMA] * 2,
    )(k_cache, v_cache)

    k_out, v_out = pl.pallas_call(
        _patch_body,
        out_shape=(out_sd, out_sd),
        in_specs=[
            pl.BlockSpec(memory_space=pltpu.SMEM),
            pl.BlockSpec(memory_space=pl.ANY),
            pl.BlockSpec(memory_space=pl.ANY),
            pl.BlockSpec(memory_space=pl.ANY),
            pl.BlockSpec(memory_space=pl.ANY),
        ],
        out_specs=(
            pl.BlockSpec(memory_space=pl.ANY),
            pl.BlockSpec(memory_space=pl.ANY),
        ),
        scratch_shapes=[pltpu.SemaphoreType.DMA] * 2,
        input_output_aliases={3: 0, 4: 1},
    )(pos, k_val, v_val, k_bulk, v_bulk)
    return (k_out, v_out)


# speedup vs baseline: 35.0619x; 35.0619x over previous
import jax
import jax.numpy as jnp
from jax.experimental import pallas as pl
from jax.experimental.pallas import tpu as pltpu

_BASE_LEN = 2048
_CH = 258
_NBUF = 8
_DEPTH = 4
_LAG = 3


def _tc_ring_body(kc, vc, ko, vo, *scratch):
    bufs = scratch[:_NBUF]
    in_sems = scratch[_NBUF:2 * _NBUF]
    out_sems = scratch[2 * _NBUF:3 * _NBUF]
    bs = ko.shape[0]
    out_len = ko.shape[1]
    n_per_b = out_len // _CH

    items = []
    for src, dst in ((kc, ko), (vc, vo)):
        for b in range(bs):
            for i in range(n_per_b):
                items.append((src, dst, b, i * _CH))
    T = len(items)

    def start_in(t):
        src, _, b, r0 = items[t]
        return pltpu.make_async_copy(
            src.at[b, pl.ds(r0, _CH)], bufs[t % _NBUF], in_sems[t % _NBUF]
        )

    def start_out(t):
        _, dst, b, r0 = items[t]
        return pltpu.make_async_copy(
            bufs[t % _NBUF], dst.at[b, pl.ds(r0, _CH)], out_sems[t % _NBUF]
        )

    in_h = [None] * T
    out_h = [None] * T
    for t in range(min(_DEPTH, T)):
        in_h[t] = start_in(t)
        in_h[t].start()
    for t in range(T):
        if t - _LAG >= 0:
            out_h[t - _LAG].wait()
        nxt = t + _DEPTH
        if nxt < T:
            in_h[nxt] = start_in(nxt)
            in_h[nxt].start()
        in_h[t].wait()
        out_h[t] = start_out(t)
        out_h[t].start()
    for t in range(max(0, T - _LAG), T):
        out_h[t].wait()


def _patch_body(pos_ref, kv, vv, _ka, _va, ko, vo, sk, sv):
    seq = kv.shape[1]
    pos = pos_ref[0]
    ck = pltpu.make_async_copy(kv, ko.at[:, pl.ds(pos, seq)], sk)
    cv = pltpu.make_async_copy(vv, vo.at[:, pl.ds(pos, seq)], sv)
    ck.start()
    cv.start()
    ck.wait()
    cv.wait()


def kernel(k_cache, v_cache, input_pos, k_val, v_val):
    bs, seq, n_heads, head_dim = k_val.shape
    out_len = _BASE_LEN + seq
    pos = jnp.asarray(input_pos, dtype=jnp.int32).reshape(1)
    out_sd = jax.ShapeDtypeStruct((bs, out_len, n_heads, head_dim), k_cache.dtype)

    k_bulk, v_bulk = pl.pallas_call(
        _tc_ring_body,
        out_shape=(out_sd, out_sd),
        in_specs=[
            pl.BlockSpec(memory_space=pl.ANY),
            pl.BlockSpec(memory_space=pl.ANY),
        ],
        out_specs=(
            pl.BlockSpec(memory_space=pl.ANY),
            pl.BlockSpec(memory_space=pl.ANY),
        ),
        scratch_shapes=(
            [pltpu.VMEM((_CH, n_heads, head_dim), k_cache.dtype)
             for _ in range(_NBUF)]
            + [pltpu.SemaphoreType.DMA] * (2 * _NBUF)
        ),
    )(k_cache, v_cache)

    k_out, v_out = pl.pallas_call(
        _patch_body,
        out_shape=(out_sd, out_sd),
        in_specs=[
            pl.BlockSpec(memory_space=pltpu.SMEM),
            pl.BlockSpec(memory_space=pl.ANY),
            pl.BlockSpec(memory_space=pl.ANY),
            pl.BlockSpec(memory_space=pl.ANY),
            pl.BlockSpec(memory_space=pl.ANY),
        ],
        out_specs=(
            pl.BlockSpec(memory_space=pl.ANY),
            pl.BlockSpec(memory_space=pl.ANY),
        ),
        scratch_shapes=[pltpu.SemaphoreType.DMA] * 2,
        input_output_aliases={3: 0, 4: 1},
    )(pos, k_val, v_val, k_bulk, v_bulk)
    return (k_out, v_out)
